# Initial kernel scaffold; baseline (speedup 1.0000x reference)
import jax
import jax.numpy as jnp
from jax.experimental import pallas as pl


def kernel(x, pe):
    raise NotImplementedError("write your pallas kernel here")


# speedup vs baseline: 3.0559x; 3.0559x over previous
import functools

import jax
import jax.numpy as jnp
from jax import lax
from jax.experimental import pallas as pl
from jax.experimental.pallas import tpu as pltpu
from jax.experimental.pallas import tpu_sc as plsc

_NUM_CORES = 2
_NUM_SUBCORES = 16
_NUM_WORKERS = _NUM_CORES * _NUM_SUBCORES


def _pe_broadcast(pe, batch, seq_len, chunk):
    embed_dim = pe.shape[1]
    rows_per_w = seq_len // _NUM_WORKERS
    n_chunks = rows_per_w // chunk
    mesh = plsc.VectorSubcoreMesh(
        core_axis_name="c",
        subcore_axis_name="s",
        num_cores=_NUM_CORES,
        num_subcores=_NUM_SUBCORES,
    )

    @functools.partial(
        pl.kernel,
        out_type=jax.ShapeDtypeStruct((batch, seq_len, embed_dim), pe.dtype),
        mesh=mesh,
        scratch_types=[
            pltpu.VMEM((chunk, embed_dim), pe.dtype),
        ],
    )
    def broadcast_kernel(pe_hbm, out_hbm, buf):
        wid = lax.axis_index("s") * _NUM_CORES + lax.axis_index("c")
        row0 = wid * rows_per_w
        for c in range(n_chunks):
            base = row0 + c * chunk
            pltpu.sync_copy(pe_hbm.at[pl.ds(base, chunk)], buf)
            for b in range(batch):
                pltpu.sync_copy(buf, out_hbm.at[b, pl.ds(base, chunk)])

    return broadcast_kernel


def kernel(x, pe):
    batch, seq_len = x.shape[0], x.shape[1]
    return _pe_broadcast(pe, batch, seq_len, chunk=64)(pe)
